# Initial kernel scaffold; baseline (speedup 1.0000x reference)
import jax
import jax.numpy as jnp
from jax.experimental import pallas as pl


def kernel(feats, segment_ids, num_segments):
    raise NotImplementedError("write your pallas kernel here")


# speedup vs baseline: 4.6473x; 4.6473x over previous
import functools

import jax
import jax.numpy as jnp
from jax import lax
from jax.experimental import pallas as pl
from jax.experimental.pallas import tpu as pltpu
from jax.experimental.pallas import tpu_sc as plsc

N = 50000
D = 256
G = 128

NCORES = 2
NTILES = 16
DC = D // NCORES
ROWS_PER_TILE = 3128
OVERLAP = NTILES * ROWS_PER_TILE - N
CHUNK = 128
NFULL = ROWS_PER_TILE // CHUNK
TAIL = ROWS_PER_TILE - NFULL * CHUNK
NCHUNK = NFULL + 1


def _body(feats_hbm, ids_hbm, out_hbm, ids_v, fbuf, zbuf, acc):
    cid = lax.axis_index("c")
    sid = lax.axis_index("s")
    col0 = cid * DC

    zero = jnp.zeros((16,), jnp.float32)
    for r in range(8):
        for j in range(DC // 16):
            zbuf[r, pl.ds(j * 16, 16)] = zero
    pltpu.sync_copy(zbuf, acc.at[pl.ds(sid * 8, 8)])

    pltpu.sync_copy(ids_hbm.at[sid], ids_v)
    plsc.subcore_barrier()

    base = jnp.minimum(sid * ROWS_PER_TILE, N - ROWS_PER_TILE)

    def chunk_body(j, carry):
        pltpu.sync_copy(
            feats_hbm.at[pl.ds(base + j * CHUNK, CHUNK), pl.ds(col0, DC)],
            fbuf,
        )
        pltpu.sync_copy(fbuf, acc.at[ids_v.at[j]], add=True)
        return carry

    lax.fori_loop(0, NFULL, chunk_body, 0)

    pltpu.sync_copy(
        feats_hbm.at[pl.ds(base + NFULL * CHUNK, TAIL), pl.ds(col0, DC)],
        fbuf.at[pl.ds(0, TAIL)],
    )
    pltpu.sync_copy(fbuf, acc.at[ids_v.at[NFULL]], add=True)

    plsc.subcore_barrier()

    pltpu.sync_copy(
        acc.at[pl.ds(sid * 8, 8)],
        out_hbm.at[pl.ds(sid * 8, 8), pl.ds(col0, DC)],
    )


def kernel(feats, segment_ids, num_segments):
    ids = segment_ids.astype(jnp.int32) + (
        jnp.asarray(num_segments, jnp.int32) - G
    )
    bases = jnp.minimum(
        jnp.arange(NTILES, dtype=jnp.int32) * ROWS_PER_TILE, N - ROWS_PER_TILE
    )
    idx = bases[:, None] + jnp.arange(ROWS_PER_TILE, dtype=jnp.int32)[None, :]
    ids = jnp.take(ids, idx, axis=0)
    ids = ids.at[NTILES - 1, :OVERLAP].set(G)
    ids = jnp.pad(
        ids,
        ((0, 0), (0, NCHUNK * CHUNK - ROWS_PER_TILE)),
        constant_values=G,
    )
    ids = ids.reshape(NTILES, NCHUNK, CHUNK)

    mesh = plsc.VectorSubcoreMesh(core_axis_name="c", subcore_axis_name="s")
    run = functools.partial(
        pl.kernel,
        mesh=mesh,
        out_type=jax.ShapeDtypeStruct((G, D), jnp.float32),
        scratch_types=[
            pltpu.VMEM((NCHUNK, CHUNK), jnp.int32),
            pltpu.VMEM((CHUNK, DC), jnp.float32),
            pltpu.VMEM((8, DC), jnp.float32),
            pltpu.VMEM_SHARED((G + 8, DC), jnp.float32),
        ],
    )(_body)
    return run(feats, ids)


# speedup vs baseline: 5.3050x; 1.1415x over previous
import functools

import jax
import jax.numpy as jnp
from jax import lax
from jax.experimental import pallas as pl
from jax.experimental.pallas import tpu as pltpu
from jax.experimental.pallas import tpu_sc as plsc

N = 50000
D = 256
G = 128

NCORES = 2
NTILES = 16
DC = D // NCORES
ROWS_PER_TILE = 3128
OVERLAP = NTILES * ROWS_PER_TILE - N
CHUNK = 128
NFULL = ROWS_PER_TILE // CHUNK
TAIL = ROWS_PER_TILE - NFULL * CHUNK
NCHUNK = NFULL + 1


def _body(feats_hbm, ids_hbm, out_hbm, ids_v, fbuf, zbuf, acc, sem0, sem1):
    cid = lax.axis_index("c")
    sid = lax.axis_index("s")
    col0 = cid * DC
    base = jnp.minimum(sid * ROWS_PER_TILE, N - ROWS_PER_TILE)
    sems = (sem0, sem1)

    def gather(j, b):
        return pltpu.make_async_copy(
            feats_hbm.at[pl.ds(base + j * CHUNK, CHUNK), pl.ds(col0, DC)],
            fbuf.at[b],
            sems[b],
        )

    gather(0, 0).start()
    gather(1, 1).start()

    zero = jnp.zeros((16,), jnp.float32)
    for r in range(8):
        for j in range(DC // 16):
            zbuf[r, pl.ds(j * 16, 16)] = zero
    pltpu.sync_copy(zbuf, acc.at[pl.ds(sid * 8, 8)])

    pltpu.sync_copy(ids_hbm.at[sid], ids_v)
    plsc.subcore_barrier()

    def pair_body(k, carry):
        for b in range(2):
            j = 2 * k + b
            gather(j, b).wait()
            pltpu.sync_copy(fbuf.at[b], acc.at[ids_v.at[j]], add=True)

            @pl.when(j + 2 < NFULL)
            def _():
                gather(j + 2, b).start()

        return carry

    lax.fori_loop(0, NFULL // 2, pair_body, 0)

    pltpu.sync_copy(
        feats_hbm.at[pl.ds(base + NFULL * CHUNK, TAIL), pl.ds(col0, DC)],
        fbuf.at[0, pl.ds(0, TAIL)],
    )
    pltpu.sync_copy(fbuf.at[0], acc.at[ids_v.at[NFULL]], add=True)

    plsc.subcore_barrier()

    pltpu.sync_copy(
        acc.at[pl.ds(sid * 8, 8)],
        out_hbm.at[pl.ds(sid * 8, 8), pl.ds(col0, DC)],
    )


def kernel(feats, segment_ids, num_segments):
    ids = segment_ids.astype(jnp.int32) + (
        jnp.asarray(num_segments, jnp.int32) - G
    )
    bases = jnp.minimum(
        jnp.arange(NTILES, dtype=jnp.int32) * ROWS_PER_TILE, N - ROWS_PER_TILE
    )
    idx = bases[:, None] + jnp.arange(ROWS_PER_TILE, dtype=jnp.int32)[None, :]
    ids = jnp.take(ids, idx, axis=0)
    ids = ids.at[NTILES - 1, :OVERLAP].set(G)
    ids = jnp.pad(
        ids,
        ((0, 0), (0, NCHUNK * CHUNK - ROWS_PER_TILE)),
        constant_values=G,
    )
    ids = ids.reshape(NTILES, NCHUNK, CHUNK)

    mesh = plsc.VectorSubcoreMesh(core_axis_name="c", subcore_axis_name="s")
    run = functools.partial(
        pl.kernel,
        mesh=mesh,
        out_type=jax.ShapeDtypeStruct((G, D), jnp.float32),
        scratch_types=[
            pltpu.VMEM((NCHUNK, CHUNK), jnp.int32),
            pltpu.VMEM((2, CHUNK, DC), jnp.float32),
            pltpu.VMEM((8, DC), jnp.float32),
            pltpu.VMEM_SHARED((G + 8, DC), jnp.float32),
            pltpu.SemaphoreType.DMA,
            pltpu.SemaphoreType.DMA,
        ],
    )(_body)
    return run(feats, ids)


# speedup vs baseline: 6.2429x; 1.1768x over previous
import functools

import jax
import jax.numpy as jnp
from jax import lax
from jax.experimental import pallas as pl
from jax.experimental.pallas import tpu as pltpu
from jax.experimental.pallas import tpu_sc as plsc

N = 50000
D = 256
G = 128

NCORES = 2
NTILES = 16
DC = D // NCORES
ROWS_PER_TILE = 3128
OVERLAP = NTILES * ROWS_PER_TILE - N
CHUNK = 128
NFULL = ROWS_PER_TILE // CHUNK
TAIL = ROWS_PER_TILE - NFULL * CHUNK
NCHUNK = NFULL + 1


def _body(feats_hbm, ids_hbm, out_hbm, ids_v, fbuf, zbuf, acc, sem0, sem1):
    cid = lax.axis_index("c")
    sid = lax.axis_index("s")
    col0 = cid * DC
    base = jnp.minimum(sid * ROWS_PER_TILE, N - ROWS_PER_TILE)
    sems = (sem0, sem1)

    def gather(j, b):
        return pltpu.make_async_copy(
            feats_hbm.at[pl.ds(base + j * CHUNK, CHUNK), pl.ds(col0, DC)],
            fbuf.at[b],
            sems[b],
        )

    gather(0, 0).start()
    gather(1, 1).start()

    zero = jnp.zeros((16,), jnp.float32)
    for r in range(8):
        for j in range(DC // 16):
            zbuf[r, pl.ds(j * 16, 16)] = zero
    pltpu.sync_copy(zbuf, acc.at[pl.ds(sid * 8, 8)])

    pltpu.sync_copy(ids_hbm.at[sid], ids_v)
    plsc.subcore_barrier()

    def pair_body(k, carry):
        for b in range(2):
            j = 2 * k + b
            gather(j, b).wait()
            pltpu.sync_copy(fbuf.at[b], acc.at[ids_v.at[j]], add=True)

            @pl.when(j + 2 < NFULL)
            def _():
                gather(j + 2, b).start()

        return carry

    lax.fori_loop(0, NFULL // 2, pair_body, 0)

    pltpu.sync_copy(
        feats_hbm.at[pl.ds(base + NFULL * CHUNK, TAIL), pl.ds(col0, DC)],
        fbuf.at[0, pl.ds(0, TAIL)],
    )
    pltpu.sync_copy(fbuf.at[0], acc.at[ids_v.at[NFULL]], add=True)

    plsc.subcore_barrier()

    pltpu.sync_copy(
        acc.at[pl.ds(sid * 8, 8)],
        out_hbm.at[pl.ds(sid * 8, 8), pl.ds(col0, DC)],
    )


def kernel(feats, segment_ids, num_segments):
    ids = segment_ids.astype(jnp.int32) + (
        jnp.asarray(num_segments, jnp.int32) - G
    )
    main = ids[: (NTILES - 1) * ROWS_PER_TILE].reshape(NTILES - 1, ROWS_PER_TILE)
    last = ids[N - ROWS_PER_TILE :]
    last = jnp.where(
        jnp.arange(ROWS_PER_TILE, dtype=jnp.int32) < OVERLAP, G, last
    )
    ids = jnp.concatenate([main, last[None]], axis=0)
    ids = jnp.pad(
        ids,
        ((0, 0), (0, NCHUNK * CHUNK - ROWS_PER_TILE)),
        constant_values=G,
    )
    ids = ids.reshape(NTILES, NCHUNK, CHUNK)

    mesh = plsc.VectorSubcoreMesh(core_axis_name="c", subcore_axis_name="s")
    run = functools.partial(
        pl.kernel,
        mesh=mesh,
        out_type=jax.ShapeDtypeStruct((G, D), jnp.float32),
        scratch_types=[
            pltpu.VMEM((NCHUNK, CHUNK), jnp.int32),
            pltpu.VMEM((2, CHUNK, DC), jnp.float32),
            pltpu.VMEM((8, DC), jnp.float32),
            pltpu.VMEM_SHARED((G + 8, DC), jnp.float32),
            pltpu.SemaphoreType.DMA,
            pltpu.SemaphoreType.DMA,
        ],
    )(_body)
    return run(feats, ids)
